# baseline (speedup 1.0000x reference)
import functools

import jax
import jax.numpy as jnp
from jax import lax
from jax.experimental import pallas as pl
from jax.experimental.pallas import tpu as pltpu
from jax.experimental.pallas import tpu_sc as plsc

N_NODES = 10000
N_EDGES = 320000
CH = 128

NC = 2
NS = 16
NW = NC * NS

K = 128
R = 10240
RPT = R // NS
CHUNKS = 80
HALF = CHUNKS // 2
EW = CHUNKS * K
E_PAD = EW * NW


def _seg_sum_body(x_hbm, src_hbm, dst_hbm, out_hbm,
                  srcb, dstb, dummyb, rows0, rows1, acc, g0, g1, s0, s1):
    cid = lax.axis_index("c")
    sid = lax.axis_index("s")
    wid = sid * NC + cid

    zero16 = jnp.zeros((16,), jnp.float32)
    dummy16 = jnp.full((16,), R - 1, jnp.int32)

    def zfill(i, _):
        r = i // (CH // 16)
        c = (i % (CH // 16)) * 16
        rows0[r, pl.ds(c, 16)] = zero16
        return 0

    lax.fori_loop(0, K * (CH // 16), zfill, 0)
    for j in range(K // 16):
        dummyb[pl.ds(j * 16, 16)] = dummy16

    def zcopy(i, _):
        pltpu.sync_copy(rows0, acc.at[pl.ds(sid * RPT + i * K, K)])
        return 0

    lax.fori_loop(0, RPT // K, zcopy, 0)
    plsc.subcore_barrier()

    def prime_scatters():
        pltpu.async_copy(rows0, acc.at[dummyb], s0, add=True)
        pltpu.async_copy(rows1, acc.at[dummyb], s1, add=True)

    def drain_scatters():
        pltpu.make_async_copy(x_hbm.at[srcb.at[0]], rows0, s0).wait()
        pltpu.make_async_copy(x_hbm.at[srcb.at[1]], rows1, s1).wait()

    def pair(i, _):
        t0 = 2 * i
        t1 = t0 + 1
        pltpu.make_async_copy(x_hbm.at[srcb.at[t0]], rows0, s0).wait()
        pltpu.async_copy(x_hbm.at[srcb.at[t0]], rows0, g0)
        pltpu.make_async_copy(x_hbm.at[srcb.at[t1]], rows1, s1).wait()
        pltpu.async_copy(x_hbm.at[srcb.at[t1]], rows1, g1)
        pltpu.make_async_copy(x_hbm.at[srcb.at[t0]], rows0, g0).wait()
        pltpu.async_copy(rows0, acc.at[dstb.at[t0]], s0, add=True)
        pltpu.make_async_copy(x_hbm.at[srcb.at[t1]], rows1, g1).wait()
        pltpu.async_copy(rows1, acc.at[dstb.at[t1]], s1, add=True)
        return 0

    for h in range(2):
        pltpu.sync_copy(src_hbm.at[wid, h], srcb)
        pltpu.sync_copy(dst_hbm.at[wid, h], dstb)
        prime_scatters()
        lax.fori_loop(0, HALF // 2, pair, 0)
        drain_scatters()
    plsc.subcore_barrier()

    pltpu.sync_copy(acc.at[pl.ds(sid * RPT, RPT)],
                    out_hbm.at[cid, pl.ds(sid * RPT, RPT)])


_seg_sum = functools.partial(
    pl.kernel,
    out_type=jax.ShapeDtypeStruct((NC, R, CH), jnp.float32),
    mesh=plsc.VectorSubcoreMesh(core_axis_name="c", subcore_axis_name="s"),
    scratch_types=[
        pltpu.VMEM((HALF, K), jnp.int32),
        pltpu.VMEM((HALF, K), jnp.int32),
        pltpu.VMEM((K,), jnp.int32),
        pltpu.VMEM((K, CH), jnp.float32),
        pltpu.VMEM((K, CH), jnp.float32),
        pltpu.VMEM_SHARED((R, CH), jnp.float32),
        pltpu.SemaphoreType.DMA,
        pltpu.SemaphoreType.DMA,
        pltpu.SemaphoreType.DMA,
        pltpu.SemaphoreType.DMA,
    ],
)(_seg_sum_body)


BLK = 1000


def _mlp_body(last, x_ref, p_ref, wa_ref, ba_ref, wb_ref, bb_ref, o_ref):
    h0 = x_ref[...] + p_ref[0] + p_ref[1]
    dn = (((1,), (1,)), ((), ()))
    t = lax.dot_general(h0, wa_ref[...], dimension_numbers=dn,
                        precision=lax.Precision.HIGHEST,
                        preferred_element_type=jnp.float32) + ba_ref[...]
    t = jnp.maximum(t, 0.0)
    h = lax.dot_general(t, wb_ref[...], dimension_numbers=dn,
                        precision=lax.Precision.HIGHEST,
                        preferred_element_type=jnp.float32) + bb_ref[...]
    if last:
        m = jnp.max(h, axis=1, keepdims=True)
        lse = jnp.log(jnp.sum(jnp.exp(h - m), axis=1, keepdims=True)) + m
        o_ref[...] = h - lse
    else:
        o_ref[...] = jnp.maximum(h, 0.0)


def _mlp(last, x, parts, wa, ba, wb, bb):
    grid = (N_NODES // BLK,)
    return pl.pallas_call(
        functools.partial(_mlp_body, last),
        grid=grid,
        in_specs=[
            pl.BlockSpec((BLK, CH), lambda i: (i, 0)),
            pl.BlockSpec((NC, BLK, CH), lambda i: (0, i, 0)),
            pl.BlockSpec((CH, CH), lambda i: (0, 0)),
            pl.BlockSpec((1, CH), lambda i: (0, 0)),
            pl.BlockSpec((CH, CH), lambda i: (0, 0)),
            pl.BlockSpec((1, CH), lambda i: (0, 0)),
        ],
        out_specs=pl.BlockSpec((BLK, CH), lambda i: (i, 0)),
        out_shape=jax.ShapeDtypeStruct((N_NODES, CH), jnp.float32),
    )(x, parts, wa, ba, wb, bb)


def kernel(x, edge_index, W1a, b1a, W1b, b1b, W2a, b2a, W2b, b2b):
    pad = E_PAD - N_EDGES
    src = jnp.concatenate([edge_index[0], jnp.zeros((pad,), jnp.int32)])
    dst = jnp.concatenate([edge_index[1], jnp.full((pad,), N_NODES, jnp.int32)])
    src = src.reshape(NW, 2, HALF, K)
    dst = dst.reshape(NW, 2, HALF, K)
    b1a2, b1b2 = b1a.reshape(1, CH), b1b.reshape(1, CH)
    b2a2, b2b2 = b2a.reshape(1, CH), b2b.reshape(1, CH)

    parts1 = _seg_sum(x, src, dst)
    h = _mlp(False, x, parts1, W1a, b1a2, W1b, b1b2)
    parts2 = _seg_sum(h, src, dst)
    return _mlp(True, h, parts2, W2a, b2a2, W2b, b2b2)


# speedup vs baseline: 1.2226x; 1.2226x over previous
import functools

import jax
import jax.numpy as jnp
from jax import lax
from jax.experimental import pallas as pl
from jax.experimental.pallas import tpu as pltpu
from jax.experimental.pallas import tpu_sc as plsc

N_NODES = 10000
N_EDGES = 320000
CH = 128

NC = 2
NS = 16
NW = NC * NS

K = 128
R = 10240
RPT = R // NS
CPS = 40
SA = 3
SB = 1
CROW = (SA + SB) * CPS
E_PAD = NS * CROW * K


def _seg_sum_body(x_hbm, src_hbm, dst_hbm, out_hbm,
                  srcb, dstb, dummyb, rows0, rows1, acc, g0, g1, s0, s1):
    cid = lax.axis_index("c")
    sid = lax.axis_index("s")

    zero16 = jnp.zeros((16,), jnp.float32)
    dummy16 = jnp.full((16,), R - 1, jnp.int32)

    def zfill(i, _):
        r = i // (CH // 16)
        c = (i % (CH // 16)) * 16
        rows0[r, pl.ds(c, 16)] = zero16
        return 0

    lax.fori_loop(0, K * (CH // 16), zfill, 0)
    for j in range(K // 16):
        dummyb[pl.ds(j * 16, 16)] = dummy16

    def zcopy(i, _):
        pltpu.sync_copy(rows0, acc.at[pl.ds(sid * RPT + i * K, K)])
        return 0

    lax.fori_loop(0, RPT // K, zcopy, 0)
    plsc.subcore_barrier()

    def prime_scatters():
        pltpu.async_copy(rows0, acc.at[dummyb], s0, add=True)
        pltpu.async_copy(rows1, acc.at[dummyb], s1, add=True)

    def drain_scatters():
        pltpu.make_async_copy(x_hbm.at[srcb.at[0]], rows0, s0).wait()
        pltpu.make_async_copy(x_hbm.at[srcb.at[1]], rows1, s1).wait()

    def pair(i, _):
        t0 = 2 * i
        t1 = t0 + 1
        pltpu.make_async_copy(x_hbm.at[srcb.at[t0]], rows0, s0).wait()
        pltpu.async_copy(x_hbm.at[srcb.at[t0]], rows0, g0)
        pltpu.make_async_copy(x_hbm.at[srcb.at[t1]], rows1, s1).wait()
        pltpu.async_copy(x_hbm.at[srcb.at[t1]], rows1, g1)
        pltpu.make_async_copy(x_hbm.at[srcb.at[t0]], rows0, g0).wait()
        pltpu.async_copy(rows0, acc.at[dstb.at[t0]], s0, add=True)
        pltpu.make_async_copy(x_hbm.at[srcb.at[t1]], rows1, g1).wait()
        pltpu.async_copy(rows1, acc.at[dstb.at[t1]], s1, add=True)
        return 0

    base_off = jnp.where(cid == 0, 0, SA * CPS)
    for g in range(SA):
        @pl.when(jnp.logical_or(cid == 0, g < SB))
        def _stage():
            off = base_off + g * CPS
            pltpu.sync_copy(src_hbm.at[sid, pl.ds(off, CPS)], srcb)
            pltpu.sync_copy(dst_hbm.at[sid, pl.ds(off, CPS)], dstb)
            prime_scatters()
            lax.fori_loop(0, CPS // 2, pair, 0)
            drain_scatters()
    plsc.subcore_barrier()

    pltpu.sync_copy(acc.at[pl.ds(sid * RPT, RPT)],
                    out_hbm.at[cid, pl.ds(sid * RPT, RPT)])


_seg_sum = functools.partial(
    pl.kernel,
    out_type=jax.ShapeDtypeStruct((NC, R, CH), jnp.float32),
    mesh=plsc.VectorSubcoreMesh(core_axis_name="c", subcore_axis_name="s"),
    scratch_types=[
        pltpu.VMEM((CPS, K), jnp.int32),
        pltpu.VMEM((CPS, K), jnp.int32),
        pltpu.VMEM((K,), jnp.int32),
        pltpu.VMEM((K, CH), jnp.float32),
        pltpu.VMEM((K, CH), jnp.float32),
        pltpu.VMEM_SHARED((R, CH), jnp.float32),
        pltpu.SemaphoreType.DMA,
        pltpu.SemaphoreType.DMA,
        pltpu.SemaphoreType.DMA,
        pltpu.SemaphoreType.DMA,
    ],
)(_seg_sum_body)


BLK = 1000


def _mlp_body(last, x_ref, p_ref, wa_ref, ba_ref, wb_ref, bb_ref, o_ref):
    h0 = x_ref[...] + p_ref[0] + p_ref[1]
    dn = (((1,), (1,)), ((), ()))
    t = lax.dot_general(h0, wa_ref[...], dimension_numbers=dn,
                        precision=lax.Precision.HIGHEST,
                        preferred_element_type=jnp.float32) + ba_ref[...]
    t = jnp.maximum(t, 0.0)
    h = lax.dot_general(t, wb_ref[...], dimension_numbers=dn,
                        precision=lax.Precision.HIGHEST,
                        preferred_element_type=jnp.float32) + bb_ref[...]
    if last:
        m = jnp.max(h, axis=1, keepdims=True)
        lse = jnp.log(jnp.sum(jnp.exp(h - m), axis=1, keepdims=True)) + m
        o_ref[...] = h - lse
    else:
        o_ref[...] = jnp.maximum(h, 0.0)


def _mlp(last, x, parts, wa, ba, wb, bb):
    grid = (N_NODES // BLK,)
    return pl.pallas_call(
        functools.partial(_mlp_body, last),
        grid=grid,
        in_specs=[
            pl.BlockSpec((BLK, CH), lambda i: (i, 0)),
            pl.BlockSpec((NC, BLK, CH), lambda i: (0, i, 0)),
            pl.BlockSpec((CH, CH), lambda i: (0, 0)),
            pl.BlockSpec((1, CH), lambda i: (0, 0)),
            pl.BlockSpec((CH, CH), lambda i: (0, 0)),
            pl.BlockSpec((1, CH), lambda i: (0, 0)),
        ],
        out_specs=pl.BlockSpec((BLK, CH), lambda i: (i, 0)),
        out_shape=jax.ShapeDtypeStruct((N_NODES, CH), jnp.float32),
    )(x, parts, wa, ba, wb, bb)


def kernel(x, edge_index, W1a, b1a, W1b, b1b, W2a, b2a, W2b, b2b):
    pad = E_PAD - N_EDGES
    src = jnp.concatenate([edge_index[0], jnp.zeros((pad,), jnp.int32)])
    dst = jnp.concatenate([edge_index[1], jnp.full((pad,), N_NODES, jnp.int32)])
    src = src.reshape(NS, CROW, K)
    dst = dst.reshape(NS, CROW, K)
    b1a2, b1b2 = b1a.reshape(1, CH), b1b.reshape(1, CH)
    b2a2, b2b2 = b2a.reshape(1, CH), b2b.reshape(1, CH)

    parts1 = _seg_sum(x, src, dst)
    h = _mlp(False, x, parts1, W1a, b1a2, W1b, b1b2)
    parts2 = _seg_sum(h, src, dst)
    return _mlp(True, h, parts2, W2a, b2a2, W2b, b2b2)
